# Initial kernel scaffold; baseline (speedup 1.0000x reference)
import jax
import jax.numpy as jnp
from jax.experimental import pallas as pl


def kernel(X, y, enc_fc1_w, enc_fc1_b, enc_fc2_w, enc_fc2_b, ln_gamma, ln_beta, enc_fc3_w, enc_fc3_b, cap_fc1_w, cap_fc1_b, cap_fc2_w, cap_fc2_b):
    raise NotImplementedError("write your pallas kernel here")


# speedup vs baseline: 136.9468x; 136.9468x over previous
import functools

import jax
import jax.numpy as jnp
import numpy as np
from jax.experimental import pallas as pl
from jax.experimental.pallas import tpu as pltpu

_D = 10
_NC = 10
_NSUB = 2 ** _D - 1
_B = 1024
_K = 16
_TEMP = 0.1

_BM = 8
_BN = 256
_BR = 128


def _incl_t():
    masks = np.arange(1, 2 ** _D, dtype=np.int64)
    incl = (masks[None, :] & masks[:, None]) == masks[None, :]
    out = np.zeros((1024, 1024), np.float32)
    out[:_NSUB, :_NSUB] = incl.astype(np.float32).T
    return jnp.asarray(out)


def _prep_kernel(x_ref, y_ref, w1_ref, b1_ref, w2_ref, b2_ref, g_ref, be_ref,
                 w3_ref, b3_ref, cw1_ref, cb1_ref, cw2_ref, cb2_ref, incl_ref,
                 m_ref, caps_ref, oh_ref):
    x = x_ref[...]
    h = jnp.maximum(jnp.dot(x, w1_ref[...], preferred_element_type=jnp.float32)
                    + b1_ref[...], 0.0)
    h = jnp.maximum(jnp.dot(h, w2_ref[...], preferred_element_type=jnp.float32)
                    + b2_ref[...], 0.0)
    mu = jnp.mean(h, axis=-1, keepdims=True)
    var = jnp.mean((h - mu) ** 2, axis=-1, keepdims=True)
    h = (h - mu) * jax.lax.rsqrt(var + 1e-5) * g_ref[...] + be_ref[...]
    h = jnp.dot(h, w3_ref[...], preferred_element_type=jnp.float32) + b3_ref[...]
    latent = jnp.mean(h, axis=0, keepdims=True)
    gg = jnp.maximum(jnp.dot(latent, cw1_ref[...],
                             preferred_element_type=jnp.float32) + cb1_ref[...], 0.0)
    raw = jnp.dot(gg, cw2_ref[...], preferred_element_type=jnp.float32) + cb2_ref[...]
    inc = jax.nn.sigmoid(raw) * 0.1
    z = jnp.sum(inc)
    m_ref[...] = inc / z
    caps = jnp.dot(inc, incl_ref[...], preferred_element_type=jnp.float32)
    caps_ref[...] = caps / z
    y = y_ref[...]
    lbl = jax.lax.broadcasted_iota(jnp.int32, (_B, _NC), 1)
    oh_ref[...] = (y == lbl).astype(jnp.float32)


def _sims_kernel(m_ref, xa_ref, xbt_ref, o_ref):
    v = []
    for f in range(_D):
        a = xa_ref[:, f:f + 1]
        b = xbt_ref[f:f + 1, :]
        v.append(jnp.exp(-jnp.abs(a - b)))
    acc = [jnp.zeros((_BM, _BN), jnp.float32)]

    def rec(parent_min, mask, start):
        for k in range(start, _D):
            nm = mask | (1 << k)
            cur = v[k] if parent_min is None else jnp.minimum(parent_min, v[k])
            acc[0] = acc[0] + m_ref[nm - 1] * cur
            rec(cur, nm, k + 1)

    rec(None, 0, 0)
    o_ref[...] = acc[0]


def _vote_kernel(sims_ref, oh_ref, o_ref):
    i = pl.program_id(0)
    s = sims_ref[...]
    col = jax.lax.broadcasted_iota(jnp.int32, (_BR, _B), 1)
    row = jax.lax.broadcasted_iota(jnp.int32, (_BR, _B), 0) + i * _BR
    s = jnp.where(col == row, -1e9, s)
    work = s
    vals = []
    for _ in range(_K):
        mt = jnp.max(work, axis=1, keepdims=True)
        vals.append(mt)
        work = jnp.where(work == mt, -jnp.inf, work)
    vmax = vals[0]
    vmin = vals[_K - 1]
    denom = jnp.zeros_like(vmax)
    for t in range(_K):
        denom = denom + jnp.exp((vals[t] - vmax) / _TEMP)
    w = jnp.where(s >= vmin, jnp.exp((s - vmax) / _TEMP), 0.0) / denom
    o_ref[...] = jnp.dot(w, oh_ref[...], preferred_element_type=jnp.float32)


@functools.partial(jax.jit, static_argnames=())
def kernel(X, y, enc_fc1_w, enc_fc1_b, enc_fc2_w, enc_fc2_b, ln_gamma, ln_beta,
           enc_fc3_w, enc_fc3_b, cap_fc1_w, cap_fc1_b, cap_fc2_w, cap_fc2_b):
    incl_t = _incl_t()
    cw2 = jnp.zeros((32, 1024), jnp.float32).at[:, :_NSUB].set(cap_fc2_w)
    cb2 = jnp.full((1, 1024), -1e9, jnp.float32).at[0, :_NSUB].set(cap_fc2_b)

    m, caps, oh = pl.pallas_call(
        _prep_kernel,
        out_shape=(
            jax.ShapeDtypeStruct((1, 1024), jnp.float32),
            jax.ShapeDtypeStruct((1, 1024), jnp.float32),
            jax.ShapeDtypeStruct((_B, _NC), jnp.float32),
        ),
    )(X, y.reshape(_B, 1), enc_fc1_w, enc_fc1_b.reshape(1, -1),
      enc_fc2_w, enc_fc2_b.reshape(1, -1), ln_gamma.reshape(1, -1),
      ln_beta.reshape(1, -1), enc_fc3_w, enc_fc3_b.reshape(1, -1),
      cap_fc1_w, cap_fc1_b.reshape(1, -1), cw2, cb2, incl_t)

    sims = pl.pallas_call(
        _sims_kernel,
        grid=(_B // _BM, _B // _BN),
        in_specs=[
            pl.BlockSpec(memory_space=pltpu.SMEM),
            pl.BlockSpec((_BM, _D), lambda i, j: (i, 0)),
            pl.BlockSpec((_D, _BN), lambda i, j: (0, j)),
        ],
        out_specs=pl.BlockSpec((_BM, _BN), lambda i, j: (i, j)),
        out_shape=jax.ShapeDtypeStruct((_B, _B), jnp.float32),
    )(m.reshape(1024), X, X.T)

    votes = pl.pallas_call(
        _vote_kernel,
        grid=(_B // _BR,),
        in_specs=[
            pl.BlockSpec((_BR, _B), lambda i: (i, 0)),
            pl.BlockSpec((_B, _NC), lambda i: (0, 0)),
        ],
        out_specs=pl.BlockSpec((_BR, _NC), lambda i: (i, 0)),
        out_shape=jax.ShapeDtypeStruct((_B, _NC), jnp.float32),
    )(sims, oh)

    return votes, caps.reshape(1024)[:_NSUB]
